# Initial kernel scaffold; baseline (speedup 1.0000x reference)
import jax
import jax.numpy as jnp
from jax.experimental import pallas as pl


def kernel(feats, indices, stride, W_pre, g_pre, b_pre, Wconv, W_pos, g_norm, b_norm, g_local, b_local):
    raise NotImplementedError("write your pallas kernel here")


# speedup vs baseline: 1.8556x; 1.8556x over previous
import functools

import numpy as np
import jax
import jax.numpy as jnp
from jax import lax
from jax.experimental import pallas as pl
from jax.experimental.pallas import tpu as pltpu
from jax.experimental.pallas import tpu_sc as plsc

INC = 64
N0 = 100000
NP = 102400
NPE = NP + 16
BLK = 1024
NBLK = NP // BLK
PROW = NP // 128

T1 = 1 << 22
SDUMP = T1
QDUMP = T1 + 8
TD = 16 * 264192
EMPTY = NP
MINR = 5

GRID = 65536
GR = 66048
CDUMP = GRID + 8
CH = 144
NCC = CH // 16

_OFFS = np.stack(
    np.meshgrid(np.arange(-1, 2), np.arange(-1, 2), np.arange(-1, 2),
                indexing="ij"), -1).reshape(-1, 3)
_HI = jax.lax.Precision.HIGHEST

_MESH1 = plsc.VectorSubcoreMesh(core_axis_name="c", subcore_axis_name="s",
                                num_cores=1)
_SC_PARAMS = pltpu.CompilerParams(use_tc_tiling_on_sc=False)
_MESH2 = plsc.VectorSubcoreMesh(core_axis_name="c", subcore_axis_name="s",
                                num_cores=2)

_I16 = lambda: lax.iota(jnp.int32, 16)


def _ln(x, g, b):
    m = jnp.mean(x, axis=1, keepdims=True)
    v = jnp.mean((x - m) ** 2, axis=1, keepdims=True)
    return (x - m) / jnp.sqrt(v + 1e-6) * g + b


def _k1_body(sref, feats, xyzi, wpt, wpreT, gpre, bpre, big, ps, pc, cc):
    i = pl.program_id(0)
    F = jnp.dot(feats[...], wpreT[...], precision=_HI)
    F = _ln(F, gpre[...], bpre[...])
    xyz = xyzi[...]
    xf = xyz[:, 0:1].astype(jnp.float32)
    yf = xyz[:, 1:2].astype(jnp.float32)
    zf = xyz[:, 2:3].astype(jnp.float32)
    wp = wpt[...]
    pw = xf * wp[0:1, :] + yf * wp[1:2, :] + zf * wp[2:3, :]
    pwt = jnp.concatenate([pw[:, : INC // 2], pw[:, : INC // 2]], 1)
    psv = jnp.sin(pwt)
    pcv = jnp.cos(pwt)
    row = i * BLK + lax.broadcasted_iota(jnp.int32, (BLK, 1), 0)
    valid = row < N0
    bigv = jnp.concatenate(
        [F * pcv, F * psv, jnp.ones((BLK, 1), jnp.float32),
         jnp.zeros((BLK, CH - 2 * INC - 1), jnp.float32)], 1)
    big[...] = jnp.where(valid, bigv, 0.0)
    ps[...] = psv
    pc[...] = pcv
    s = sref[0, 0]
    ccv = ((xyz[:, 3:4] << 15) + ((xyz[:, 0:1] // s) << 10)
           + ((xyz[:, 1:2] // s) << 5) + (xyz[:, 2:3] // s))
    cc[...] = jnp.where(valid, ccv, CDUMP)


def _k1(strides, feats_p, xyzi, wpt, wpreT, gpre, bpre):
    return pl.pallas_call(
        _k1_body,
        grid=(NBLK,),
        in_specs=[
            pl.BlockSpec(memory_space=pltpu.SMEM),
            pl.BlockSpec((BLK, INC), lambda i: (i, 0)),
            pl.BlockSpec((BLK, 4), lambda i: (i, 0)),
            pl.BlockSpec((8, INC), lambda i: (0, 0)),
            pl.BlockSpec((INC, INC), lambda i: (0, 0)),
            pl.BlockSpec((1, INC), lambda i: (0, 0)),
            pl.BlockSpec((1, INC), lambda i: (0, 0)),
        ],
        out_specs=[
            pl.BlockSpec((BLK, CH), lambda i: (i, 0)),
            pl.BlockSpec((BLK, INC), lambda i: (i, 0)),
            pl.BlockSpec((BLK, INC), lambda i: (i, 0)),
            pl.BlockSpec((BLK, 1), lambda i: (i, 0)),
        ],
        out_shape=[
            jax.ShapeDtypeStruct((NP, CH), jnp.float32),
            jax.ShapeDtypeStruct((NP, INC), jnp.float32),
            jax.ShapeDtypeStruct((NP, INC), jnp.float32),
            jax.ShapeDtypeStruct((NP, 1), jnp.int32),
        ],
    )(strides, feats_p, xyzi, wpt, wpreT, gpre, bpre)


def _sc_build_body(xh, yh, zh, bh, table, xv, yv, zv, bv,
                   key2, val2, w2, sk2, ebuf, sem):
    sid = lax.axis_index("s")

    def fill(j, _):
        ebuf[pl.ds(j * 16, 16)] = jnp.full((16,), EMPTY, jnp.int32)
        return ()
    lax.fori_loop(0, 128, fill, ())

    tbase = sid * 264192

    def init(c, _):
        pltpu.sync_copy(ebuf, table.at[pl.ds(tbase + c * 2048, 2048)])
        return ()
    lax.fori_loop(0, 129, init, ())
    plsc.subcore_barrier()

    base = sid * 6400
    pltpu.sync_copy(xh.at[pl.ds(base, 6400)], xv)
    pltpu.sync_copy(yh.at[pl.ds(base, 6400)], yv)
    pltpu.sync_copy(zh.at[pl.ds(base, 6400)], zv)
    pltpu.sync_copy(bh.at[pl.ds(base, 6400)], bv)

    def keys(j, _):
        c = j // 8
        r = j - c * 8
        sl = pl.ds(j * 16, 16)
        x16 = xv[sl]
        k16 = ((bv[sl] << 21) + (x16 << 14) + (yv[sl] << 7) + zv[sl])
        pos = base + j * 16 + _I16()
        k16 = jnp.where(pos < N0, k16, SDUMP)
        key2[c, pl.ds(r * 16, 16)] = k16
        val2[c, pl.ds(r * 16, 16)] = pos
        return ()
    lax.fori_loop(0, 400, keys, ())

    def fire_sc(c, _):
        pltpu.async_copy(val2.at[c], table.at[key2.at[c]], sem)
        return ()

    def drain_sc(c, _):
        pltpu.make_async_copy(val2.at[c], table.at[key2.at[c]], sem).wait()
        return ()

    lax.fori_loop(0, 50, fire_sc, ())
    lax.fori_loop(0, 50, drain_sc, ())
    plsc.subcore_barrier()

    for _ in range(MINR):
        def fire_g(c, _):
            pltpu.async_copy(table.at[key2.at[c]], w2.at[c], sem)
            return ()

        def drain_g(c, _):
            pltpu.make_async_copy(table.at[key2.at[c]], w2.at[c], sem).wait()
            return ()

        lax.fori_loop(0, 50, fire_g, ())
        lax.fori_loop(0, 50, drain_g, ())

        def mk(j, _):
            c = j // 8
            r = j - c * 8
            sl = pl.ds(r * 16, 16)
            v16 = val2[c, sl]
            sk2[c, sl] = jnp.where(v16 < w2[c, sl], key2[c, sl], SDUMP)
            return ()
        lax.fori_loop(0, 400, mk, ())

        def fire_s(c, _):
            pltpu.async_copy(val2.at[c], table.at[sk2.at[c]], sem)
            return ()

        def drain_s(c, _):
            pltpu.make_async_copy(val2.at[c], table.at[sk2.at[c]], sem).wait()
            return ()

        lax.fori_loop(0, 50, fire_s, ())
        lax.fori_loop(0, 50, drain_s, ())
        plsc.subcore_barrier()


_sc_build = pl.kernel(
    _sc_build_body,
    out_type=jax.ShapeDtypeStruct((TD,), jnp.int32),
    mesh=_MESH1,
    compiler_params=_SC_PARAMS,
    scratch_types=[
        pltpu.VMEM((6400,), jnp.int32), pltpu.VMEM((6400,), jnp.int32),
        pltpu.VMEM((6400,), jnp.int32), pltpu.VMEM((6400,), jnp.int32),
        pltpu.VMEM((50, 128), jnp.int32), pltpu.VMEM((50, 128), jnp.int32),
        pltpu.VMEM((50, 128), jnp.int32), pltpu.VMEM((50, 128), jnp.int32),
        pltpu.VMEM((2048,), jnp.int32),
        pltpu.SemaphoreType.DMA,
    ],
)


def _sc_idx_body(xh, yh, zh, bh, table, idxall, xv, yv, zv, bv,
                 kv, fv, nk2, idv, sem):
    wid = lax.axis_index("s") * 2 + lax.axis_index("c")
    base = wid * 3200
    pltpu.sync_copy(xh.at[pl.ds(base, 3200)], xv)
    pltpu.sync_copy(yh.at[pl.ds(base, 3200)], yv)
    pltpu.sync_copy(zh.at[pl.ds(base, 3200)], zv)
    pltpu.sync_copy(bh.at[pl.ds(base, 3200)], bv)

    def keys(j, _):
        sl = pl.ds(j * 16, 16)
        x16 = xv[sl]
        y16 = yv[sl]
        z16 = zv[sl]
        kv[sl] = (bv[sl] << 21) + (x16 << 14) + (y16 << 7) + z16
        one = jnp.full((16,), 1, jnp.int32)
        zero = jnp.zeros((16,), jnp.int32)
        f16 = (jnp.where(x16 == 0, one, zero)
               + jnp.where(x16 == 127, 2, 0)
               + jnp.where(y16 == 0, 4, 0)
               + jnp.where(y16 == 127, 8, 0)
               + jnp.where(z16 == 0, 16, 0)
               + jnp.where(z16 == 127, 32, 0))
        fv[sl] = f16
        return ()
    lax.fori_loop(0, 200, keys, ())

    for k in range(27):
        dx, dy, dz = (int(v) for v in _OFFS[k])
        dk = (dx << 14) + (dy << 7) + dz
        mk = ((1 if dx < 0 else (2 if dx > 0 else 0))
              | (4 if dy < 0 else (8 if dy > 0 else 0))
              | (16 if dz < 0 else (32 if dz > 0 else 0)))

        def nkey(j, _, dk=dk, mk=mk):
            c = j // 8
            r = j - c * 8
            sl = pl.ds(j * 16, 16)
            bad = (fv[sl] & mk) != 0
            nk2[c, pl.ds(r * 16, 16)] = jnp.where(bad, QDUMP, kv[sl] + dk)
            return ()
        lax.fori_loop(0, 200, nkey, ())

        def fire(c, _):
            pltpu.async_copy(table.at[nk2.at[c]],
                             idv.at[pl.ds(c * 128, 128)], sem)
            return ()

        def drain(c, _):
            pltpu.make_async_copy(table.at[nk2.at[c]],
                                  idv.at[pl.ds(c * 128, 128)], sem).wait()
            return ()

        lax.fori_loop(0, 25, fire, ())
        lax.fori_loop(0, 25, drain, ())
        pltpu.sync_copy(idv, idxall.at[pl.ds(k * NP + base, 3200)])


_sc_idx = pl.kernel(
    _sc_idx_body,
    out_type=jax.ShapeDtypeStruct((27 * NP,), jnp.int32),
    mesh=_MESH2,
    compiler_params=_SC_PARAMS,
    scratch_types=[
        pltpu.VMEM((3200,), jnp.int32), pltpu.VMEM((3200,), jnp.int32),
        pltpu.VMEM((3200,), jnp.int32), pltpu.VMEM((3200,), jnp.int32),
        pltpu.VMEM((3200,), jnp.int32), pltpu.VMEM((3200,), jnp.int32),
        pltpu.VMEM((25, 128), jnp.int32), pltpu.VMEM((3200,), jnp.int32),
        pltpu.SemaphoreType.DMA,
    ],
)


def _sc_gather_body(idxall, fext, gout, idv, rows8, semg, semw):
    wid = lax.axis_index("s") * 2 + lax.axis_index("c")
    base = wid * 3200

    for k in range(27):
        pltpu.sync_copy(idxall.at[pl.ds(k * NP + base, 3200)], idv)
        for c in range(4):
            pltpu.async_copy(fext.at[idv.at[pl.ds(c * 128, 128)]],
                             rows8.at[c], semg)

        def step(c, _, k=k):
            slot = lax.rem(c, 8)
            pltpu.make_async_copy(fext.at[idv.at[pl.ds(c * 128, 128)]],
                                  rows8.at[slot], semg).wait()
            pltpu.async_copy(rows8.at[slot],
                             gout.at[k, pl.ds(base + c * 128, 128), :], semw)

            @pl.when(c + 4 < 25)
            def _():
                @pl.when(c >= 4)
                def _():
                    pltpu.make_async_copy(
                        rows8.at[0], gout.at[k, pl.ds(base, 128), :],
                        semw).wait()
                pltpu.async_copy(
                    fext.at[idv.at[pl.ds((c + 4) * 128, 128)]],
                    rows8.at[lax.rem(c + 4, 8)], semg)
            return ()
        lax.fori_loop(0, 25, step, ())

        def draw(j, _, k=k):
            pltpu.make_async_copy(rows8.at[0],
                                  gout.at[k, pl.ds(base, 128), :], semw).wait()
            return ()
        lax.fori_loop(0, 8, draw, ())


_sc_gather = pl.kernel(
    _sc_gather_body,
    out_type=jax.ShapeDtypeStruct((27, NP, INC), jnp.float32),
    mesh=_MESH2,
    compiler_params=_SC_PARAMS,
    scratch_types=[
        pltpu.VMEM((3200,), jnp.int32),
        pltpu.VMEM((8, 128, INC), jnp.float32),
        pltpu.SemaphoreType.DMA, pltpu.SemaphoreType.DMA,
    ],
)


def _sc_vox_body(cch, bigh, gridout, cv2, rrows, zer, gbuf, semr, sema):
    cid = lax.axis_index("c")
    sid = lax.axis_index("s")
    pbase = cid * (NP // 2) + sid * 3200
    gbase = sid * (GR // 16)

    def ldcv(c, _):
        pltpu.sync_copy(cch.at[pl.ds(pbase + c * 128, 128)], cv2.at[c])
        return ()
    lax.fori_loop(0, 25, ldcv, ())

    def zfill(j, _):
        zer[j, :] = jnp.zeros((16,), jnp.float32)
        return ()
    lax.fori_loop(0, 1032, zfill, ())

    for cc in range(NCC):
        def init(i, _):
            pltpu.sync_copy(zer, gbuf.at[pl.ds(gbase + i * 1032, 1032), :])
            return ()
        lax.fori_loop(0, 4, init, ())
        plsc.subcore_barrier()

        for c in range(2):
            pltpu.async_copy(
                bigh.at[pl.ds(pbase + c * 128, 128), pl.ds(cc * 16, 16)],
                rrows.at[c], semr)

        def step(c, _, cc=cc):
            slot = lax.rem(c, 4)
            pltpu.make_async_copy(
                bigh.at[pl.ds(pbase + c * 128, 128), pl.ds(cc * 16, 16)],
                rrows.at[slot], semr).wait()
            pltpu.async_copy(rrows.at[slot], gbuf.at[cv2.at[c]], sema,
                             add=True)

            @pl.when(c + 2 < 25)
            def _():
                @pl.when(c >= 2)
                def _():
                    pltpu.make_async_copy(rrows.at[0],
                                          gbuf.at[pl.ds(0, 128), :],
                                          sema).wait()
                pltpu.async_copy(
                    bigh.at[pl.ds(pbase + (c + 2) * 128, 128),
                            pl.ds(cc * 16, 16)],
                    rrows.at[lax.rem(c + 2, 4)], semr)
            return ()
        lax.fori_loop(0, 25, step, ())

        def dradd(j, _):
            pltpu.make_async_copy(rrows.at[0], gbuf.at[pl.ds(0, 128), :],
                                  sema).wait()
            return ()
        lax.fori_loop(0, 4, dradd, ())
        plsc.subcore_barrier()

        pltpu.sync_copy(
            gbuf.at[pl.ds(gbase, GR // 16), :],
            gridout.at[cid, pl.ds(gbase, GR // 16), pl.ds(cc * 16, 16)])
        plsc.subcore_barrier()


_sc_vox = pl.kernel(
    _sc_vox_body,
    out_type=jax.ShapeDtypeStruct((2, GR, CH), jnp.float32),
    mesh=_MESH2,
    compiler_params=_SC_PARAMS,
    scratch_types=[
        pltpu.VMEM((25, 128), jnp.int32),
        pltpu.VMEM((4, 128, 16), jnp.float32),
        pltpu.VMEM((1032, 16), jnp.float32),
        pltpu.VMEM_SHARED((GR, 16), jnp.float32),
        pltpu.SemaphoreType.DMA, pltpu.SemaphoreType.DMA,
    ],
)


def _zy_pass(g, czc, cyc):
    up = jnp.concatenate([g[1:], jnp.zeros((1, CH), jnp.float32)], 0)
    dn = jnp.concatenate([jnp.zeros((1, CH), jnp.float32), g[:-1]], 0)
    t = g + jnp.where(czc < 31, up, 0.0) + jnp.where(czc > 0, dn, 0.0)
    up = jnp.concatenate([t[32:], jnp.zeros((32, CH), jnp.float32)], 0)
    dn = jnp.concatenate([jnp.zeros((32, CH), jnp.float32), t[:-32]], 0)
    return t + jnp.where(cyc < 31, up, 0.0) + jnp.where(cyc > 0, dn, 0.0)


def _k2_body(c0c, c1c, c0u, c1u, c0d, c1d, out):
    s = pl.program_id(0)
    rr = lax.broadcasted_iota(jnp.int32, (BLK, 1), 0)
    czc = lax.rem(rr, 32)
    cyc = rr // 32
    uc = _zy_pass(c0c[0] + c1c[0], czc, cyc)
    uu = _zy_pass(c0u[0] + c1u[0], czc, cyc)
    ud = _zy_pass(c0d[0] + c1d[0], czc, cyc)
    cx = lax.rem(s, 32)
    w = (uc + jnp.where(cx < 31, uu, 0.0) + jnp.where(cx > 0, ud, 0.0))
    denom = w[:, 2 * INC:2 * INC + 1]
    denom = jnp.where(denom != 0.0, denom, 1.0)
    out[...] = w[:, :2 * INC] / denom


def _k2(gridc):
    bs = lambda f: pl.BlockSpec((1, BLK, CH), f)
    return pl.pallas_call(
        _k2_body,
        grid=(64,),
        in_specs=[
            bs(lambda s: (0, s, 0)), bs(lambda s: (1, s, 0)),
            bs(lambda s: (0, jnp.minimum(s + 1, 63), 0)),
            bs(lambda s: (1, jnp.minimum(s + 1, 63), 0)),
            bs(lambda s: (0, jnp.maximum(s - 1, 0), 0)),
            bs(lambda s: (1, jnp.maximum(s - 1, 0), 0)),
        ],
        out_specs=pl.BlockSpec((BLK, 2 * INC), lambda s: (s, 0)),
        out_shape=jax.ShapeDtypeStruct((GR, 2 * INC), jnp.float32),
    )(gridc, gridc, gridc, gridc, gridc, gridc)


def _sc_devox_body(cch, fcell, lout, cv, rows4, semg, semw):
    wid = lax.axis_index("s") * 2 + lax.axis_index("c")
    base = wid * 3200
    pltpu.sync_copy(cch.at[pl.ds(base, 3200)], cv)

    for c in range(2):
        pltpu.async_copy(fcell.at[cv.at[pl.ds(c * 128, 128)]],
                         rows4.at[c], semg)

    def step(c, _):
        slot = lax.rem(c, 4)
        pltpu.make_async_copy(fcell.at[cv.at[pl.ds(c * 128, 128)]],
                              rows4.at[slot], semg).wait()
        pltpu.async_copy(rows4.at[slot],
                         lout.at[pl.ds(base + c * 128, 128), :], semw)

        @pl.when(c + 2 < 25)
        def _():
            @pl.when(c >= 2)
            def _():
                pltpu.make_async_copy(rows4.at[0],
                                      lout.at[pl.ds(base, 128), :],
                                      semw).wait()
            pltpu.async_copy(fcell.at[cv.at[pl.ds((c + 2) * 128, 128)]],
                             rows4.at[lax.rem(c + 2, 4)], semg)
        return ()
    lax.fori_loop(0, 25, step, ())

    def draw(j, _):
        pltpu.make_async_copy(rows4.at[0], lout.at[pl.ds(base, 128), :],
                              semw).wait()
        return ()
    lax.fori_loop(0, 4, draw, ())


_sc_devox = pl.kernel(
    _sc_devox_body,
    out_type=jax.ShapeDtypeStruct((NP, 2 * INC), jnp.float32),
    mesh=_MESH2,
    compiler_params=_SC_PARAMS,
    scratch_types=[
        pltpu.VMEM((3200,), jnp.int32),
        pltpu.VMEM((4, 128, 2 * INC), jnp.float32),
        pltpu.SemaphoreType.DMA, pltpu.SemaphoreType.DMA,
    ],
)


def _k3_body(g, wc, lf, ps, pc, gn, bn, gl, bl, out):
    acc = jnp.dot(g[0], wc[0], precision=_HI)
    for k in range(1, 27):
        acc = acc + jnp.dot(g[k], wc[k], precision=_HI)
    loc = _ln(acc, gl[...], bl[...])
    lfv = lf[...]
    of = lfv[:, :INC] * pc[...] + lfv[:, INC:] * ps[...]
    of = _ln(of, gn[...], bn[...])
    out[...] = jnp.maximum(of + loc, 0.0)


def _k3(gmat, wconv, largef, ps, pc, gn, bn, gl, bl):
    return pl.pallas_call(
        _k3_body,
        grid=(NBLK,),
        in_specs=[
            pl.BlockSpec((27, BLK, INC), lambda i: (0, i, 0)),
            pl.BlockSpec((27, INC, INC), lambda i: (0, 0, 0)),
            pl.BlockSpec((BLK, 2 * INC), lambda i: (i, 0)),
            pl.BlockSpec((BLK, INC), lambda i: (i, 0)),
            pl.BlockSpec((BLK, INC), lambda i: (i, 0)),
            pl.BlockSpec((1, INC), lambda i: (0, 0)),
            pl.BlockSpec((1, INC), lambda i: (0, 0)),
            pl.BlockSpec((1, INC), lambda i: (0, 0)),
            pl.BlockSpec((1, INC), lambda i: (0, 0)),
        ],
        out_specs=pl.BlockSpec((BLK, INC), lambda i: (i, 0)),
        out_shape=jax.ShapeDtypeStruct((NP, INC), jnp.float32),
    )(gmat, wconv, largef, ps, pc, gn, bn, gl, bl)


def kernel(feats, indices, stride, W_pre, g_pre, b_pre, Wconv, W_pos,
           g_norm, b_norm, g_local, b_local):
    N = feats.shape[0]
    pad = NP - N
    xh = jnp.pad(indices[:, 3], (0, pad))
    yh = jnp.pad(indices[:, 2], (0, pad))
    zh = jnp.pad(indices[:, 1], (0, pad))
    bh = jnp.pad(indices[:, 0], (0, pad))
    xyzi = jnp.pad(indices[:, jnp.array([3, 2, 1, 0])], ((0, pad), (0, 0)))
    feats_p = jnp.pad(feats, ((0, pad), (0, 0)))
    fext = jnp.pad(feats, ((0, NPE - N), (0, 0)))
    wpt = jnp.pad(W_pos.T, ((0, 5), (0, 0)))
    strides = jnp.asarray(stride, jnp.int32).reshape(1, 1)

    big, ps, pc, cc1 = _k1(strides, feats_p, xyzi, wpt, W_pre.T,
                           g_pre.reshape(1, INC), b_pre.reshape(1, INC))
    cc2 = cc1.reshape(NP)

    table = _sc_build(xh, yh, zh, bh)
    idxall = _sc_idx(xh, yh, zh, bh, table)
    gmat = _sc_gather(idxall, fext)

    gridc = _sc_vox(cc2, big)
    fcell = _k2(gridc)
    largef = _sc_devox(cc2, fcell)

    out = _k3(gmat, Wconv, largef, ps, pc,
              g_norm.reshape(1, INC), b_norm.reshape(1, INC),
              g_local.reshape(1, INC), b_local.reshape(1, INC))
    return out[:N], indices


# speedup vs baseline: 1.8590x; 1.0018x over previous
import jax
import jax.numpy as jnp
from jax import lax
from jax.experimental import pallas as pl
from jax.experimental.pallas import tpu as pltpu
from jax.experimental.pallas import tpu_sc as plsc

INC = 64
N0 = 100000
NP = 102400
NPE = NP + 16
BLK = 1024
NBLK = NP // BLK

T1 = 1 << 22
SDUMP = T1
QDUMP = T1 + 8
TD = 16 * 264192
EMPTY = NP
MINR = 5

GRID = 65536
GR = 66048
CDUMP = GRID + 8
CH = 144
NCC = CH // 16

_HI = jax.lax.Precision.HIGHEST

_MESH1 = plsc.VectorSubcoreMesh(core_axis_name="c", subcore_axis_name="s",
                                num_cores=1)
_MESH2 = plsc.VectorSubcoreMesh(core_axis_name="c", subcore_axis_name="s",
                                num_cores=2)
_SC_PARAMS = pltpu.CompilerParams(use_tc_tiling_on_sc=False)

_I16 = lambda: lax.iota(jnp.int32, 16)


def _ln(x, g, b):
    m = jnp.mean(x, axis=1, keepdims=True)
    v = jnp.mean((x - m) ** 2, axis=1, keepdims=True)
    return (x - m) / jnp.sqrt(v + 1e-6) * g + b


def _k1_body(sref, feats, xyzi, wpt, wpreT, gpre, bpre, big, ps, pc, cc):
    i = pl.program_id(0)
    F = jnp.dot(feats[...], wpreT[...], precision=_HI)
    F = _ln(F, gpre[...], bpre[...])
    xyz = xyzi[...]
    xf = xyz[:, 0:1].astype(jnp.float32)
    yf = xyz[:, 1:2].astype(jnp.float32)
    zf = xyz[:, 2:3].astype(jnp.float32)
    wp = wpt[...]
    pw = xf * wp[0:1, :] + yf * wp[1:2, :] + zf * wp[2:3, :]
    pwt = jnp.concatenate([pw[:, : INC // 2], pw[:, : INC // 2]], 1)
    psv = jnp.sin(pwt)
    pcv = jnp.cos(pwt)
    row = i * BLK + lax.broadcasted_iota(jnp.int32, (BLK, 1), 0)
    valid = row < N0
    bigv = jnp.concatenate(
        [F * pcv, F * psv, jnp.ones((BLK, 1), jnp.float32),
         jnp.zeros((BLK, CH - 2 * INC - 1), jnp.float32)], 1)
    big[...] = jnp.where(valid, bigv, 0.0)
    ps[...] = psv
    pc[...] = pcv
    s = sref[0, 0]
    ccv = ((xyz[:, 3:4] << 15) + ((xyz[:, 0:1] // s) << 10)
           + ((xyz[:, 1:2] // s) << 5) + (xyz[:, 2:3] // s))
    cc[...] = jnp.where(valid, ccv, CDUMP)


def _k1(strides, feats_p, xyzi, wpt, wpreT, gpre, bpre):
    return pl.pallas_call(
        _k1_body,
        grid=(NBLK,),
        in_specs=[
            pl.BlockSpec(memory_space=pltpu.SMEM),
            pl.BlockSpec((BLK, INC), lambda i: (i, 0)),
            pl.BlockSpec((BLK, 4), lambda i: (i, 0)),
            pl.BlockSpec((8, INC), lambda i: (0, 0)),
            pl.BlockSpec((INC, INC), lambda i: (0, 0)),
            pl.BlockSpec((1, INC), lambda i: (0, 0)),
            pl.BlockSpec((1, INC), lambda i: (0, 0)),
        ],
        out_specs=[
            pl.BlockSpec((BLK, CH), lambda i: (i, 0)),
            pl.BlockSpec((BLK, INC), lambda i: (i, 0)),
            pl.BlockSpec((BLK, INC), lambda i: (i, 0)),
            pl.BlockSpec((BLK, 1), lambda i: (i, 0)),
        ],
        out_shape=[
            jax.ShapeDtypeStruct((NP, CH), jnp.float32),
            jax.ShapeDtypeStruct((NP, INC), jnp.float32),
            jax.ShapeDtypeStruct((NP, INC), jnp.float32),
            jax.ShapeDtypeStruct((NP, 1), jnp.int32),
        ],
    )(strides, feats_p, xyzi, wpt, wpreT, gpre, bpre)


def _sc_build_body(xh, yh, zh, bh, table, xv, yv, zv, bv,
                   key2, val2, w2, sk2, ebuf, sem):
    sid = lax.axis_index("s")

    def fill(j, _):
        ebuf[pl.ds(j * 16, 16)] = jnp.full((16,), EMPTY, jnp.int32)
        return ()
    lax.fori_loop(0, 128, fill, ())

    tbase = sid * 264192

    def init(c, _):
        pltpu.sync_copy(ebuf, table.at[pl.ds(tbase + c * 2048, 2048)])
        return ()
    lax.fori_loop(0, 129, init, ())
    plsc.subcore_barrier()

    base = sid * 6400
    pltpu.sync_copy(xh.at[pl.ds(base, 6400)], xv)
    pltpu.sync_copy(yh.at[pl.ds(base, 6400)], yv)
    pltpu.sync_copy(zh.at[pl.ds(base, 6400)], zv)
    pltpu.sync_copy(bh.at[pl.ds(base, 6400)], bv)

    def keys(j, _):
        c = j // 8
        r = j - c * 8
        sl = pl.ds(j * 16, 16)
        x16 = xv[sl]
        k16 = ((bv[sl] << 21) + (x16 << 14) + (yv[sl] << 7) + zv[sl])
        pos = base + j * 16 + _I16()
        k16 = jnp.where(pos < N0, k16, SDUMP)
        key2[c, pl.ds(r * 16, 16)] = k16
        val2[c, pl.ds(r * 16, 16)] = pos
        return ()
    lax.fori_loop(0, 400, keys, ())

    def fire_sc(c, _):
        pltpu.async_copy(val2.at[c], table.at[key2.at[c]], sem)
        return ()

    def drain_sc(c, _):
        pltpu.make_async_copy(val2.at[c], table.at[key2.at[c]], sem).wait()
        return ()

    lax.fori_loop(0, 50, fire_sc, ())
    lax.fori_loop(0, 50, drain_sc, ())
    plsc.subcore_barrier()

    for _ in range(MINR):
        def fire_g(c, _):
            pltpu.async_copy(table.at[key2.at[c]], w2.at[c], sem)
            return ()

        def drain_g(c, _):
            pltpu.make_async_copy(table.at[key2.at[c]], w2.at[c], sem).wait()
            return ()

        lax.fori_loop(0, 50, fire_g, ())
        lax.fori_loop(0, 50, drain_g, ())

        def mk(j, _):
            c = j // 8
            r = j - c * 8
            sl = pl.ds(r * 16, 16)
            v16 = val2[c, sl]
            sk2[c, sl] = jnp.where(v16 < w2[c, sl], key2[c, sl], SDUMP)
            return ()
        lax.fori_loop(0, 400, mk, ())

        def fire_s(c, _):
            pltpu.async_copy(val2.at[c], table.at[sk2.at[c]], sem)
            return ()

        def drain_s(c, _):
            pltpu.make_async_copy(val2.at[c], table.at[sk2.at[c]], sem).wait()
            return ()

        lax.fori_loop(0, 50, fire_s, ())
        lax.fori_loop(0, 50, drain_s, ())
        plsc.subcore_barrier()


_sc_build = pl.kernel(
    _sc_build_body,
    out_type=jax.ShapeDtypeStruct((TD,), jnp.int32),
    mesh=_MESH1,
    compiler_params=_SC_PARAMS,
    scratch_types=[
        pltpu.VMEM((6400,), jnp.int32), pltpu.VMEM((6400,), jnp.int32),
        pltpu.VMEM((6400,), jnp.int32), pltpu.VMEM((6400,), jnp.int32),
        pltpu.VMEM((50, 128), jnp.int32), pltpu.VMEM((50, 128), jnp.int32),
        pltpu.VMEM((50, 128), jnp.int32), pltpu.VMEM((50, 128), jnp.int32),
        pltpu.VMEM((2048,), jnp.int32),
        pltpu.SemaphoreType.DMA,
    ],
)


def _sc_idx_body(xh, yh, zh, bh, table, idxall, xv, yv, zv, bv,
                 kv, fv, nk2, idv, sem):
    wid = lax.axis_index("s") * 2 + lax.axis_index("c")
    base = wid * 3200
    pltpu.sync_copy(xh.at[pl.ds(base, 3200)], xv)
    pltpu.sync_copy(yh.at[pl.ds(base, 3200)], yv)
    pltpu.sync_copy(zh.at[pl.ds(base, 3200)], zv)
    pltpu.sync_copy(bh.at[pl.ds(base, 3200)], bv)

    def keys(j, _):
        sl = pl.ds(j * 16, 16)
        x16 = xv[sl]
        y16 = yv[sl]
        z16 = zv[sl]
        kv[sl] = (bv[sl] << 21) + (x16 << 14) + (y16 << 7) + z16
        one = jnp.full((16,), 1, jnp.int32)
        zero = jnp.zeros((16,), jnp.int32)
        f16 = (jnp.where(x16 == 0, one, zero)
               + jnp.where(x16 == 127, 2, 0)
               + jnp.where(y16 == 0, 4, 0)
               + jnp.where(y16 == 127, 8, 0)
               + jnp.where(z16 == 0, 16, 0)
               + jnp.where(z16 == 127, 32, 0))
        fv[sl] = f16
        return ()
    lax.fori_loop(0, 200, keys, ())

    for k in range(27):
        dx = k // 9 - 1
        dy = (k // 3) % 3 - 1
        dz = k % 3 - 1
        dk = (dx << 14) + (dy << 7) + dz
        mk = ((1 if dx < 0 else (2 if dx > 0 else 0))
              | (4 if dy < 0 else (8 if dy > 0 else 0))
              | (16 if dz < 0 else (32 if dz > 0 else 0)))

        def nkey(j, _, dk=dk, mk=mk):
            c = j // 8
            r = j - c * 8
            sl = pl.ds(j * 16, 16)
            bad = (fv[sl] & mk) != 0
            nk2[c, pl.ds(r * 16, 16)] = jnp.where(bad, QDUMP, kv[sl] + dk)
            return ()
        lax.fori_loop(0, 200, nkey, ())

        def fire(c, _):
            pltpu.async_copy(table.at[nk2.at[c]],
                             idv.at[pl.ds(c * 128, 128)], sem)
            return ()

        def drain(c, _):
            pltpu.make_async_copy(table.at[nk2.at[c]],
                                  idv.at[pl.ds(c * 128, 128)], sem).wait()
            return ()

        lax.fori_loop(0, 25, fire, ())
        lax.fori_loop(0, 25, drain, ())
        pltpu.sync_copy(idv, idxall.at[pl.ds(k * NP + base, 3200)])


_sc_idx = pl.kernel(
    _sc_idx_body,
    out_type=jax.ShapeDtypeStruct((27 * NP,), jnp.int32),
    mesh=_MESH2,
    compiler_params=_SC_PARAMS,
    scratch_types=[
        pltpu.VMEM((3200,), jnp.int32), pltpu.VMEM((3200,), jnp.int32),
        pltpu.VMEM((3200,), jnp.int32), pltpu.VMEM((3200,), jnp.int32),
        pltpu.VMEM((3200,), jnp.int32), pltpu.VMEM((3200,), jnp.int32),
        pltpu.VMEM((25, 128), jnp.int32), pltpu.VMEM((3200,), jnp.int32),
        pltpu.SemaphoreType.DMA,
    ],
)


def _sc_gather_body(idxall, fext, gout, idv, rows8, semg, semw):
    wid = lax.axis_index("s") * 2 + lax.axis_index("c")
    base = wid * 3200

    for k in range(27):
        pltpu.sync_copy(idxall.at[pl.ds(k * NP + base, 3200)], idv)
        for c in range(4):
            pltpu.async_copy(fext.at[idv.at[pl.ds(c * 128, 128)]],
                             rows8.at[c], semg)

        def step(c, _, k=k):
            slot = lax.rem(c, 8)
            pltpu.make_async_copy(fext.at[idv.at[pl.ds(c * 128, 128)]],
                                  rows8.at[slot], semg).wait()
            pltpu.async_copy(rows8.at[slot],
                             gout.at[k, pl.ds(base + c * 128, 128), :], semw)

            @pl.when(c + 4 < 25)
            def _():
                @pl.when(c >= 4)
                def _():
                    pltpu.make_async_copy(
                        rows8.at[0], gout.at[k, pl.ds(base, 128), :],
                        semw).wait()
                pltpu.async_copy(
                    fext.at[idv.at[pl.ds((c + 4) * 128, 128)]],
                    rows8.at[lax.rem(c + 4, 8)], semg)
            return ()
        lax.fori_loop(0, 25, step, ())

        def draw(j, _, k=k):
            pltpu.make_async_copy(rows8.at[0],
                                  gout.at[k, pl.ds(base, 128), :],
                                  semw).wait()
            return ()
        lax.fori_loop(0, 8, draw, ())


_sc_gather = pl.kernel(
    _sc_gather_body,
    out_type=jax.ShapeDtypeStruct((27, NP, INC), jnp.float32),
    mesh=_MESH2,
    compiler_params=_SC_PARAMS,
    scratch_types=[
        pltpu.VMEM((3200,), jnp.int32),
        pltpu.VMEM((8, 128, INC), jnp.float32),
        pltpu.SemaphoreType.DMA, pltpu.SemaphoreType.DMA,
    ],
)


def _sc_vox_body(cch, bigh, gridout, cv2, rrows, zer, gbuf, semr, sema):
    cid = lax.axis_index("c")
    sid = lax.axis_index("s")
    pbase = cid * (NP // 2) + sid * 3200
    gbase = sid * (GR // 16)

    def ldcv(c, _):
        pltpu.sync_copy(cch.at[pl.ds(pbase + c * 128, 128)], cv2.at[c])
        return ()
    lax.fori_loop(0, 25, ldcv, ())

    def zfill(j, _):
        zer[j, :] = jnp.zeros((16,), jnp.float32)
        return ()
    lax.fori_loop(0, 1032, zfill, ())

    for cc in range(NCC):
        def init(i, _):
            pltpu.sync_copy(zer, gbuf.at[pl.ds(gbase + i * 1032, 1032), :])
            return ()
        lax.fori_loop(0, 4, init, ())
        plsc.subcore_barrier()

        for c in range(2):
            pltpu.async_copy(
                bigh.at[pl.ds(pbase + c * 128, 128), pl.ds(cc * 16, 16)],
                rrows.at[c], semr)

        def step(c, _, cc=cc):
            slot = lax.rem(c, 4)
            pltpu.make_async_copy(
                bigh.at[pl.ds(pbase + c * 128, 128), pl.ds(cc * 16, 16)],
                rrows.at[slot], semr).wait()
            pltpu.async_copy(rrows.at[slot], gbuf.at[cv2.at[c]], sema,
                             add=True)

            @pl.when(c + 2 < 25)
            def _():
                @pl.when(c >= 2)
                def _():
                    pltpu.make_async_copy(rrows.at[0],
                                          gbuf.at[pl.ds(0, 128), :],
                                          sema).wait()
                pltpu.async_copy(
                    bigh.at[pl.ds(pbase + (c + 2) * 128, 128),
                            pl.ds(cc * 16, 16)],
                    rrows.at[lax.rem(c + 2, 4)], semr)
            return ()
        lax.fori_loop(0, 25, step, ())

        def dradd(j, _):
            pltpu.make_async_copy(rrows.at[0], gbuf.at[pl.ds(0, 128), :],
                                  sema).wait()
            return ()
        lax.fori_loop(0, 4, dradd, ())
        plsc.subcore_barrier()

        pltpu.sync_copy(
            gbuf.at[pl.ds(gbase, GR // 16), :],
            gridout.at[cid, pl.ds(gbase, GR // 16), pl.ds(cc * 16, 16)])
        plsc.subcore_barrier()


_sc_vox = pl.kernel(
    _sc_vox_body,
    out_type=jax.ShapeDtypeStruct((2, GR, CH), jnp.float32),
    mesh=_MESH2,
    compiler_params=_SC_PARAMS,
    scratch_types=[
        pltpu.VMEM((25, 128), jnp.int32),
        pltpu.VMEM((4, 128, 16), jnp.float32),
        pltpu.VMEM((1032, 16), jnp.float32),
        pltpu.VMEM_SHARED((GR, 16), jnp.float32),
        pltpu.SemaphoreType.DMA, pltpu.SemaphoreType.DMA,
    ],
)


def _zy_pass(g, czc, cyc):
    up = jnp.concatenate([g[1:], jnp.zeros((1, CH), jnp.float32)], 0)
    dn = jnp.concatenate([jnp.zeros((1, CH), jnp.float32), g[:-1]], 0)
    t = g + jnp.where(czc < 31, up, 0.0) + jnp.where(czc > 0, dn, 0.0)
    up = jnp.concatenate([t[32:], jnp.zeros((32, CH), jnp.float32)], 0)
    dn = jnp.concatenate([jnp.zeros((32, CH), jnp.float32), t[:-32]], 0)
    return t + jnp.where(cyc < 31, up, 0.0) + jnp.where(cyc > 0, dn, 0.0)


def _k2_body(c0c, c1c, c0u, c1u, c0d, c1d, out):
    s = pl.program_id(0)
    rr = lax.broadcasted_iota(jnp.int32, (BLK, 1), 0)
    czc = lax.rem(rr, 32)
    cyc = rr // 32
    uc = _zy_pass(c0c[0] + c1c[0], czc, cyc)
    uu = _zy_pass(c0u[0] + c1u[0], czc, cyc)
    ud = _zy_pass(c0d[0] + c1d[0], czc, cyc)
    cx = lax.rem(s, 32)
    w = (uc + jnp.where(cx < 31, uu, 0.0) + jnp.where(cx > 0, ud, 0.0))
    denom = w[:, 2 * INC:2 * INC + 1]
    denom = jnp.where(denom != 0.0, denom, 1.0)
    out[...] = w[:, :2 * INC] / denom


def _k2(gridc):
    bs = lambda f: pl.BlockSpec((1, BLK, CH), f)
    return pl.pallas_call(
        _k2_body,
        grid=(64,),
        in_specs=[
            bs(lambda s: (0, s, 0)), bs(lambda s: (1, s, 0)),
            bs(lambda s: (0, jnp.minimum(s + 1, 63), 0)),
            bs(lambda s: (1, jnp.minimum(s + 1, 63), 0)),
            bs(lambda s: (0, jnp.maximum(s - 1, 0), 0)),
            bs(lambda s: (1, jnp.maximum(s - 1, 0), 0)),
        ],
        out_specs=pl.BlockSpec((BLK, 2 * INC), lambda s: (s, 0)),
        out_shape=jax.ShapeDtypeStruct((GR, 2 * INC), jnp.float32),
    )(gridc, gridc, gridc, gridc, gridc, gridc)


def _sc_devox_body(cch, fcell, lout, cv, rows4, semg, semw):
    wid = lax.axis_index("s") * 2 + lax.axis_index("c")
    base = wid * 3200
    pltpu.sync_copy(cch.at[pl.ds(base, 3200)], cv)

    for c in range(2):
        pltpu.async_copy(fcell.at[cv.at[pl.ds(c * 128, 128)]],
                         rows4.at[c], semg)

    def step(c, _):
        slot = lax.rem(c, 4)
        pltpu.make_async_copy(fcell.at[cv.at[pl.ds(c * 128, 128)]],
                              rows4.at[slot], semg).wait()
        pltpu.async_copy(rows4.at[slot],
                         lout.at[pl.ds(base + c * 128, 128), :], semw)

        @pl.when(c + 2 < 25)
        def _():
            @pl.when(c >= 2)
            def _():
                pltpu.make_async_copy(rows4.at[0],
                                      lout.at[pl.ds(base, 128), :],
                                      semw).wait()
            pltpu.async_copy(fcell.at[cv.at[pl.ds((c + 2) * 128, 128)]],
                             rows4.at[lax.rem(c + 2, 4)], semg)
        return ()
    lax.fori_loop(0, 25, step, ())

    def draw(j, _):
        pltpu.make_async_copy(rows4.at[0], lout.at[pl.ds(base, 128), :],
                              semw).wait()
        return ()
    lax.fori_loop(0, 4, draw, ())


_sc_devox = pl.kernel(
    _sc_devox_body,
    out_type=jax.ShapeDtypeStruct((NP, 2 * INC), jnp.float32),
    mesh=_MESH2,
    compiler_params=_SC_PARAMS,
    scratch_types=[
        pltpu.VMEM((3200,), jnp.int32),
        pltpu.VMEM((4, 128, 2 * INC), jnp.float32),
        pltpu.SemaphoreType.DMA, pltpu.SemaphoreType.DMA,
    ],
)


def _k3_body(g, wc, lf, ps, pc, gn, bn, gl, bl, out):
    acc = jnp.dot(g[0], wc[0], precision=_HI)
    for k in range(1, 27):
        acc = acc + jnp.dot(g[k], wc[k], precision=_HI)
    loc = _ln(acc, gl[...], bl[...])
    lfv = lf[...]
    of = lfv[:, :INC] * pc[...] + lfv[:, INC:] * ps[...]
    of = _ln(of, gn[...], bn[...])
    out[...] = jnp.maximum(of + loc, 0.0)


def _k3(gmat, wconv, largef, ps, pc, gn, bn, gl, bl):
    return pl.pallas_call(
        _k3_body,
        grid=(NBLK,),
        in_specs=[
            pl.BlockSpec((27, BLK, INC), lambda i: (0, i, 0)),
            pl.BlockSpec((27, INC, INC), lambda i: (0, 0, 0)),
            pl.BlockSpec((BLK, 2 * INC), lambda i: (i, 0)),
            pl.BlockSpec((BLK, INC), lambda i: (i, 0)),
            pl.BlockSpec((BLK, INC), lambda i: (i, 0)),
            pl.BlockSpec((1, INC), lambda i: (0, 0)),
            pl.BlockSpec((1, INC), lambda i: (0, 0)),
            pl.BlockSpec((1, INC), lambda i: (0, 0)),
            pl.BlockSpec((1, INC), lambda i: (0, 0)),
        ],
        out_specs=pl.BlockSpec((BLK, INC), lambda i: (i, 0)),
        out_shape=jax.ShapeDtypeStruct((NP, INC), jnp.float32),
    )(gmat, wconv, largef, ps, pc, gn, bn, gl, bl)


def kernel(feats, indices, stride, W_pre, g_pre, b_pre, Wconv, W_pos,
           g_norm, b_norm, g_local, b_local):
    N = feats.shape[0]
    pad = NP - N
    xh = jnp.pad(indices[:, 3], (0, pad))
    yh = jnp.pad(indices[:, 2], (0, pad))
    zh = jnp.pad(indices[:, 1], (0, pad))
    bh = jnp.pad(indices[:, 0], (0, pad))
    xyzi = jnp.pad(indices[:, jnp.array([3, 2, 1, 0])], ((0, pad), (0, 0)))
    feats_p = jnp.pad(feats, ((0, pad), (0, 0)))
    fext = jnp.pad(feats, ((0, NPE - N), (0, 0)))
    wpt = jnp.pad(W_pos.T, ((0, 5), (0, 0)))
    strides = jnp.asarray(stride, jnp.int32).reshape(1, 1)

    big, ps, pc, cc1 = _k1(strides, feats_p, xyzi, wpt, W_pre.T,
                           g_pre.reshape(1, INC), b_pre.reshape(1, INC))
    cc2 = cc1.reshape(NP)

    table = _sc_build(xh, yh, zh, bh)
    idxall = _sc_idx(xh, yh, zh, bh, table)
    gmat = _sc_gather(idxall, fext)

    gridc = _sc_vox(cc2, big)
    fcell = _k2(gridc)
    largef = _sc_devox(cc2, fcell)

    out = _k3(gmat, Wconv, largef, ps, pc,
              g_norm.reshape(1, INC), b_norm.reshape(1, INC),
              g_local.reshape(1, INC), b_local.reshape(1, INC))
    return out[:N], indices
